# baseline (speedup 1.0000x reference)
import functools

import jax
import jax.numpy as jnp
from jax import lax
from jax.experimental import pallas as pl
from jax.experimental.pallas import tpu as pltpu
from jax.experimental.pallas import tpu_sc as plsc

VOCAB = 1000000
EMBED = 64
BATCH = 16384
POS = 10
NEG = 50

NC = 2
NS = 16
NW = NC * NS
L = 16

B_PER_W = BATCH // NW
CB = 8
STEPS = B_PER_W // CB
NEG_CHUNKS = CB * NEG // 80

O_CEN = 0
O_POS = CB
O_NEG = CB + CB * POS
O_CEN_OFF = CB * (1 + POS + NEG)
O_POS_OFF = O_CEN_OFF + CB
O_NEG_OFF = O_POS_OFF + CB * POS
PACK_W = O_CEN_OFF * 2 + L


def _sc_dots_kernel(pack_hbm, in_hbm, out_hbm, dots_hbm,
                    idx_v, cen_rows, pos_rows, neg_rows, dots_v, sem):
    wid = lax.axis_index("s") * NC + lax.axis_index("c")
    lane = lax.broadcasted_iota(jnp.int32, (L,), 0)

    def step(s, carry):
        chunk = wid * STEPS + s
        b0 = chunk * CB
        pltpu.sync_copy(pack_hbm.at[chunk], idx_v)
        cps = [pltpu.async_copy(in_hbm.at[idx_v.at[pl.ds(O_CEN, CB)]],
                                cen_rows, sem),
               pltpu.async_copy(out_hbm.at[idx_v.at[pl.ds(O_POS, CB * POS)]],
                                pos_rows, sem)]
        for k in range(NEG_CHUNKS):
            cps.append(pltpu.async_copy(
                out_hbm.at[idx_v.at[pl.ds(O_NEG + 80 * k, 80)]],
                neg_rows.at[pl.ds(80 * k, 80)], sem))
        for cp in cps:
            cp.wait()

        def item(b, carry2):
            cbase = idx_v[pl.ds(O_CEN_OFF + b, L)][0]
            c = [cen_rows[b, pl.ds(cbase + L * k, L)] for k in range(4)]
            d = [jnp.zeros((L,), jnp.float32) for _ in range(4)]
            poff = idx_v[pl.ds(O_POS_OFF + b * POS, L)]
            noff = [idx_v[pl.ds(O_NEG_OFF + b * NEG + L * t, L)]
                    for t in range(4)]
            for j in range(POS):
                row = b * POS + j
                pbase = poff[j]
                acc = pos_rows[row, pl.ds(pbase, L)] * c[0]
                for k in range(1, 4):
                    acc = acc + pos_rows[row, pl.ds(pbase + L * k, L)] * c[k]
                dot = jnp.sum(acc)
                g, ln = divmod(j, L)
                d[g] = jnp.where(lane == ln, dot, d[g])
            for j in range(NEG):
                row = b * NEG + j
                nbase = noff[j // L][j % L]
                acc = neg_rows[row, pl.ds(nbase, L)] * c[0]
                for k in range(1, 4):
                    acc = acc + neg_rows[row, pl.ds(nbase + L * k, L)] * c[k]
                dot = jnp.sum(acc)
                g, ln = divmod(POS + j, L)
                d[g] = jnp.where(lane == ln, dot, d[g])
            for g in range(4):
                dots_v[b, pl.ds(L * g, L)] = d[g]
            return carry2

        lax.fori_loop(0, CB, item, 0)
        pltpu.sync_copy(dots_v, dots_hbm.at[pl.ds(b0, CB)])
        return carry

    lax.fori_loop(0, STEPS, step, 0)


def _tc_loss_kernel(dots_ref, out_ref):
    x = dots_ref[...]
    lane = lax.broadcasted_iota(jnp.int32, x.shape, 1)
    sign = jnp.where(lane < POS, 1.0, -1.0).astype(jnp.float32)
    y = jax.nn.log_sigmoid(x * sign)
    y = jnp.where(lane < POS + NEG, y, 0.0)
    out_ref[...] = -jnp.sum(y, axis=1)


def kernel(cen_tensor, pos_tensors, neg_tensors, in_table, out_table):
    nchunk = BATCH // CB
    cen = cen_tensor.reshape(nchunk, CB)
    pos = pos_tensors.reshape(nchunk, CB * POS)
    neg = neg_tensors.reshape(nchunk, CB * NEG)
    packed = jnp.concatenate(
        [cen >> 1, pos >> 1, neg >> 1,
         (cen & 1) * EMBED, (pos & 1) * EMBED, (neg & 1) * EMBED,
         jnp.zeros((nchunk, L), jnp.int32)],
        axis=1)
    in2 = in_table.reshape(VOCAB // 2, 2 * EMBED)
    out2 = out_table.reshape(VOCAB // 2, 2 * EMBED)

    mesh = plsc.VectorSubcoreMesh(core_axis_name="c", subcore_axis_name="s")
    sc_call = functools.partial(
        pl.kernel, mesh=mesh,
        compiler_params=pltpu.CompilerParams(needs_layout_passes=False),
        out_type=jax.ShapeDtypeStruct((BATCH, EMBED), jnp.float32),
        scratch_types=[
            pltpu.VMEM((PACK_W,), jnp.int32),
            pltpu.VMEM((CB, 2 * EMBED), jnp.float32),
            pltpu.VMEM((CB * POS, 2 * EMBED), jnp.float32),
            pltpu.VMEM((CB * NEG, 2 * EMBED), jnp.float32),
            pltpu.VMEM((CB, EMBED), jnp.float32),
            pltpu.SemaphoreType.DMA,
        ],
    )(_sc_dots_kernel)
    dots = sc_call(packed, in2, out2)

    return pl.pallas_call(
        _tc_loss_kernel,
        out_shape=jax.ShapeDtypeStruct((BATCH,), jnp.float32),
    )(dots)


# speedup vs baseline: 1.2179x; 1.2179x over previous
import functools

import jax
import jax.numpy as jnp
from jax import lax
from jax.experimental import pallas as pl
from jax.experimental.pallas import tpu as pltpu
from jax.experimental.pallas import tpu_sc as plsc

VOCAB = 1000000
EMBED = 64
BATCH = 16384
POS = 10
NEG = 50

NC = 2
NS = 16
NW = NC * NS
L = 16

B_PER_W = BATCH // NW
CB = 8
STEPS = B_PER_W // CB
NEG_CHUNKS = CB * NEG // 80

O_CEN = 0
O_POS = CB
O_NEG = CB + CB * POS
PACK_W = 512
NCHUNK = BATCH // CB


def _sc_dots_kernel(pack_hbm, in_hbm, out_hbm, dots_hbm,
                    idx_v, cen_rows, pos_rows, neg_rows, dots_v, sem0, sem1):
    wid = lax.axis_index("s") * NC + lax.axis_index("c")
    lane = lax.broadcasted_iota(jnp.int32, (L,), 0)
    sems = [sem0, sem1]

    def stage(chunk, buf):
        pltpu.sync_copy(pack_hbm.at[pl.ds(chunk * PACK_W, PACK_W)],
                        idx_v.at[buf])
        sem = sems[buf]
        cps = [pltpu.async_copy(in_hbm.at[idx_v.at[buf, pl.ds(O_CEN, CB)]],
                                cen_rows.at[buf], sem),
               pltpu.async_copy(
                   out_hbm.at[idx_v.at[buf, pl.ds(O_POS, CB * POS)]],
                   pos_rows.at[buf], sem)]
        for k in range(NEG_CHUNKS):
            cps.append(pltpu.async_copy(
                out_hbm.at[idx_v.at[buf, pl.ds(O_NEG + 80 * k, 80)]],
                neg_rows.at[buf, pl.ds(80 * k, 80)], sem))
        return cps

    def drain(buf):
        sem = sems[buf]
        pltpu.make_async_copy(in_hbm.at[idx_v.at[buf, pl.ds(O_CEN, CB)]],
                              cen_rows.at[buf], sem).wait()
        pltpu.make_async_copy(out_hbm.at[idx_v.at[buf, pl.ds(O_POS, CB * POS)]],
                              pos_rows.at[buf], sem).wait()
        for k in range(NEG_CHUNKS):
            pltpu.make_async_copy(
                out_hbm.at[idx_v.at[buf, pl.ds(O_NEG + 80 * k, 80)]],
                neg_rows.at[buf, pl.ds(80 * k, 80)], sem).wait()

    def compute(s, buf):
        b0 = (wid * STEPS + s) * CB

        def item(b, carry):
            c = [cen_rows[buf, b, pl.ds(L * k, L)] for k in range(4)]
            d = [jnp.zeros((L,), jnp.float32) for _ in range(4)]
            for j in range(POS):
                row = b * POS + j
                acc = pos_rows[buf, row, pl.ds(0, L)] * c[0]
                for k in range(1, 4):
                    acc = acc + pos_rows[buf, row, pl.ds(L * k, L)] * c[k]
                dot = jnp.sum(acc)
                g, ln = divmod(j, L)
                d[g] = jnp.where(lane == ln, dot, d[g])
            for j in range(NEG):
                row = b * NEG + j
                acc = neg_rows[buf, row, pl.ds(0, L)] * c[0]
                for k in range(1, 4):
                    acc = acc + neg_rows[buf, row, pl.ds(L * k, L)] * c[k]
                dot = jnp.sum(acc)
                g, ln = divmod(POS + j, L)
                d[g] = jnp.where(lane == ln, dot, d[g])
            for g in range(4):
                dots_v[buf, pl.ds(b * EMBED + L * g, L)] = d[g]
            return carry

        lax.fori_loop(0, CB, item, 0)
        pltpu.sync_copy(dots_v.at[buf],
                        dots_hbm.at[pl.ds(b0 * EMBED, CB * EMBED)])

    stage(wid * STEPS, 0)

    def two_steps(s2, carry):
        s = s2 * 2
        stage(wid * STEPS + s + 1, 1)
        drain(0)
        compute(s, 0)

        @pl.when(s + 2 < STEPS)
        def _():
            stage(wid * STEPS + s + 2, 0)

        drain(1)
        compute(s + 1, 1)
        return carry

    lax.fori_loop(0, STEPS // 2, two_steps, 0)


def _tc_loss_kernel(dots_ref, out_ref):
    x = dots_ref[...]
    lane = lax.broadcasted_iota(jnp.int32, x.shape, 1)
    m = lax.rem(lane, EMBED)
    sign = jnp.where(m < POS, 1.0, -1.0).astype(jnp.float32)
    y = jax.nn.log_sigmoid(x * sign)
    y = jnp.where(m < POS + NEG, y, 0.0)
    s0 = -jnp.sum(y[:, :EMBED], axis=1, keepdims=True)
    s1 = -jnp.sum(y[:, EMBED:], axis=1, keepdims=True)
    out_ref[...] = jnp.concatenate([s0, s1], axis=1)


def kernel(cen_tensor, pos_tensors, neg_tensors, in_table, out_table):
    cen = cen_tensor.reshape(NCHUNK, CB)
    pos = pos_tensors.reshape(NCHUNK, CB * POS)
    neg = neg_tensors.reshape(NCHUNK, CB * NEG)
    packed = jnp.concatenate(
        [cen, pos, neg,
         jnp.zeros((NCHUNK, PACK_W - O_NEG - CB * NEG), jnp.int32)],
        axis=1).reshape(-1)

    mesh = plsc.VectorSubcoreMesh(core_axis_name="c", subcore_axis_name="s")
    sc_call = functools.partial(
        pl.kernel, mesh=mesh,
        compiler_params=pltpu.CompilerParams(needs_layout_passes=False,
                                             use_tc_tiling_on_sc=False),
        out_type=jax.ShapeDtypeStruct((BATCH * EMBED,), jnp.float32),
        scratch_types=[
            pltpu.VMEM((2, PACK_W), jnp.int32),
            pltpu.VMEM((2, CB, EMBED), jnp.float32),
            pltpu.VMEM((2, CB * POS, EMBED), jnp.float32),
            pltpu.VMEM((2, CB * NEG, EMBED), jnp.float32),
            pltpu.VMEM((2, CB * EMBED), jnp.float32),
            pltpu.SemaphoreType.DMA,
            pltpu.SemaphoreType.DMA,
        ],
    )(_sc_dots_kernel)
    dots = sc_call(packed, in_table, out_table)

    loss2 = pl.pallas_call(
        _tc_loss_kernel,
        out_shape=jax.ShapeDtypeStruct((BATCH // 2, 2), jnp.float32),
    )(dots.reshape(BATCH // 2, 2 * EMBED))
    return loss2.reshape(BATCH)
